# baseline (speedup 1.0000x reference)
import functools
import math

import jax
import jax.numpy as jnp
from jax import lax
from jax.experimental import pallas as pl
from jax.experimental.pallas import tpu as pltpu
from jax.experimental.pallas import tpu_sc as plsc

D_MODEL = 64
SCALE = math.sqrt(D_MODEL)

NC, NS, L = 2, 16, 16
NW = NC * NS

B_TOTAL = 4096 * 200
CHUNK = 128
CHUNKS_TOTAL = B_TOTAL // CHUNK
CHUNKS_PER_W = CHUNKS_TOTAL // NW
NBUF = 4
LAG = 2


def _sc_embed(x2d, lut):
    mesh = plsc.VectorSubcoreMesh(core_axis_name="c", subcore_axis_name="s")

    @functools.partial(
        pl.kernel,
        mesh=mesh,
        out_type=jax.ShapeDtypeStruct((B_TOTAL, D_MODEL), jnp.float32),
        scratch_types=[
            pltpu.VMEM((CHUNKS_PER_W, CHUNK), jnp.int32),
            pltpu.VMEM((NBUF * CHUNK, D_MODEL), jnp.float32),
        ]
        + [pltpu.SemaphoreType.DMA] * NBUF
        + [pltpu.SemaphoreType.DMA] * NBUF,
        compiler_params=pltpu.CompilerParams(use_tc_tiling_on_sc=False),
    )
    def k(x_hbm, lut_hbm, out_hbm, idx_v, rows_v, *sems):
        gsem = sems[:NBUF]
        ssem = sems[NBUF:]
        c = lax.axis_index("c")
        s = lax.axis_index("s")
        wid = s * NC + c
        chunk0 = wid * CHUNKS_PER_W

        pltpu.sync_copy(x_hbm.at[pl.ds(chunk0, CHUNKS_PER_W)], idx_v)

        def gather(g, b):
            return pltpu.make_async_copy(
                lut_hbm.at[idx_v.at[g]],
                rows_v.at[pl.ds(b * CHUNK, CHUNK)],
                gsem[b],
            )

        def scatter(g, b):
            return pltpu.make_async_copy(
                rows_v.at[pl.ds(b * CHUNK, CHUNK)],
                out_hbm.at[pl.ds((chunk0 + g) * CHUNK, CHUNK)],
                ssem[b],
            )

        for b in range(NBUF):
            gather(b, b).start()

        def outer(i, carry):
            g0 = i * NBUF
            for b in range(NBUF):
                g = g0 + b
                gather(g, b).wait()

                def scale_chunk(b):
                    @plsc.parallel_loop(0, CHUNK, unroll=4)
                    def _scale(r):
                        row = b * CHUNK + r
                        for d in range(D_MODEL // L):
                            sl = (row, pl.ds(d * L, L))
                            rows_v[sl] = rows_v[sl] * SCALE

                scale_chunk(b)

                scatter(g, b).start()

                bp = (b - LAG) % NBUF
                gp = g + NBUF - LAG

                @pl.when(jnp.logical_and(g >= LAG, gp < CHUNKS_PER_W))
                def _refill(g=g, b=b, bp=bp, gp=gp):
                    scatter(gp - NBUF, bp).wait()
                    gather(gp, bp).start()
            return carry

        lax.fori_loop(0, CHUNKS_PER_W // NBUF, outer, 0)

        for b in range(NBUF):
            g = CHUNKS_PER_W - NBUF + b
            scatter(g, b).wait()

    return k(x2d, lut)


def kernel(x, lut):
    x2d = x.reshape(CHUNKS_TOTAL, CHUNK)
    out = _sc_embed(x2d, lut)
    return out.reshape(4096, 200, D_MODEL)
